# Initial kernel scaffold; baseline (speedup 1.0000x reference)
import functools

import jax
import jax.numpy as jnp
from jax import lax
from jax.experimental import pallas as pl
from jax.experimental.pallas import tpu as pltpu
from jax.experimental.pallas import tpu_sc as plsc

_N = 10000
_E = 320000
_D = 128

_NC = 2
_NS = 16
_NW = _NC * _NS
_CH_SUB = 128
_SUBS = 4
_CHUNK = _SUBS * _CH_SUB
_CHUNKS = 20
_EW = _CHUNK * _CHUNKS
_E_PAD = _EW * _NW
_N_PAD = 10016
_ZROWS = _N_PAD // _NS
_OROWS = _N // _NS

_BLK = 1000


def _sc_agg(de):
  mesh = plsc.VectorSubcoreMesh(core_axis_name="c", subcore_axis_name="s")

  @functools.partial(
      pl.kernel,
      mesh=mesh,
      out_type=jax.ShapeDtypeStruct((_NC, _N, de), jnp.float32),
      scratch_types=[
          pltpu.VMEM((_CH_SUB, de), jnp.float32),
          pltpu.VMEM((_CH_SUB, de), jnp.float32),
          pltpu.VMEM((_CH_SUB, de), jnp.float32),
          pltpu.VMEM((_CH_SUB, de), jnp.float32),
          pltpu.VMEM((_SUBS, _CH_SUB), jnp.int32),
          pltpu.VMEM((_SUBS, _CH_SUB), jnp.int32),
          pltpu.VMEM_SHARED((_N_PAD, de), jnp.float32),
          pltpu.SemaphoreType.DMA,
      ],
  )
  def agg(table_h, src_h, dst_h, zeros_h, part_h,
          rows0, rows1, rows2, rows3, src_v, dst_v, acc, sem):
    rows = [rows0, rows1, rows2, rows3]
    c = lax.axis_index("c")
    s = lax.axis_index("s")
    wid = s * _NC + c

    pltpu.sync_copy(zeros_h.at[pl.ds(s * _ZROWS, _ZROWS)],
                    acc.at[pl.ds(s * _ZROWS, _ZROWS)])
    plsc.subcore_barrier()

    def body(i, carry):
      row = wid * _CHUNKS + i
      pltpu.sync_copy(src_h.at[row], src_v)
      pltpu.sync_copy(dst_h.at[row], dst_v)
      cps = [pltpu.async_copy(table_h.at[src_v.at[j]], rows[j], sem)
             for j in range(_SUBS)]
      for cp in cps:
        cp.wait()
      for j in range(_SUBS):
        pltpu.sync_copy(rows[j], acc.at[dst_v.at[j]], add=True)
      return carry

    lax.fori_loop(0, _CHUNKS, body, 0)
    plsc.subcore_barrier()

    pltpu.sync_copy(acc.at[pl.ds(s * _OROWS, _OROWS)],
                    part_h.at[c, pl.ds(s * _OROWS, _OROWS)])

  return agg


def _rep(shape):
  return pl.BlockSpec(shape, lambda i: tuple(0 for _ in shape))


def _dense_first(x, wa, ca, wb, cb):
  de, dv = wa.shape[1], wb.shape[1]

  def body(x_ref, wa_ref, ca_ref, wb_ref, cb_ref, t_ref, v_ref):
    xb = x_ref[...]
    t_ref[...] = jnp.dot(xb, wa_ref[...],
                         preferred_element_type=jnp.float32) + ca_ref[...]
    v_ref[...] = jnp.dot(xb, wb_ref[...],
                         preferred_element_type=jnp.float32) + cb_ref[...]

  return pl.pallas_call(
      body,
      grid=(_N // _BLK,),
      in_specs=[
          pl.BlockSpec((_BLK, _D), lambda i: (i, 0)),
          _rep(wa.shape), _rep(ca.shape), _rep(wb.shape), _rep(cb.shape),
      ],
      out_specs=[
          pl.BlockSpec((_BLK, de), lambda i: (i, 0)),
          pl.BlockSpec((_BLK, dv), lambda i: (i, 0)),
      ],
      out_shape=[
          jax.ShapeDtypeStruct((_N, de), jnp.float32),
          jax.ShapeDtypeStruct((_N, dv), jnp.float32),
      ],
  )(x, wa, ca, wb, cb)


def _dense_mid(p, vprev, wa, ca, wb, cb):
  din = vprev.shape[1]
  dpe = p.shape[2]
  de, dv = wa.shape[1], wb.shape[1]

  def body(p_ref, vp_ref, wa_ref, ca_ref, wb_ref, cb_ref, t_ref, v_ref):
    sall = p_ref[0] + p_ref[1]
    inv = 1.0 / jnp.maximum(sall[:, din:din + 1], 1.0)
    h = jnp.maximum(sall[:, :din] * inv + vp_ref[...], 0.0)
    t_ref[...] = jnp.dot(h, wa_ref[...],
                         preferred_element_type=jnp.float32) + ca_ref[...]
    v_ref[...] = jnp.dot(h, wb_ref[...],
                         preferred_element_type=jnp.float32) + cb_ref[...]

  return pl.pallas_call(
      body,
      grid=(_N // _BLK,),
      in_specs=[
          pl.BlockSpec((_NC, _BLK, dpe), lambda i: (0, i, 0)),
          pl.BlockSpec((_BLK, din), lambda i: (i, 0)),
          _rep(wa.shape), _rep(ca.shape), _rep(wb.shape), _rep(cb.shape),
      ],
      out_specs=[
          pl.BlockSpec((_BLK, de), lambda i: (i, 0)),
          pl.BlockSpec((_BLK, dv), lambda i: (i, 0)),
      ],
      out_shape=[
          jax.ShapeDtypeStruct((_N, de), jnp.float32),
          jax.ShapeDtypeStruct((_N, dv), jnp.float32),
      ],
  )(p, vprev, wa, ca, wb, cb)


def _dense_last(p, vprev, wrow, brow):
  din = vprev.shape[1]
  dpe = p.shape[2]

  def body(p_ref, vp_ref, w_ref, b_ref, o_ref):
    sall = p_ref[0] + p_ref[1]
    inv = 1.0 / jnp.maximum(sall[:, din:din + 1], 1.0)
    h = jnp.maximum(sall[:, :din] * inv + vp_ref[...], 0.0)
    o_ref[...] = jnp.sum(h * w_ref[...], axis=1, keepdims=True) + b_ref[...]

  return pl.pallas_call(
      body,
      grid=(_N // _BLK,),
      in_specs=[
          pl.BlockSpec((_NC, _BLK, dpe), lambda i: (0, i, 0)),
          pl.BlockSpec((_BLK, din), lambda i: (i, 0)),
          _rep(wrow.shape), _rep(brow.shape),
      ],
      out_specs=pl.BlockSpec((_BLK, 1), lambda i: (i, 0)),
      out_shape=jax.ShapeDtypeStruct((_N, 1), jnp.float32),
  )(p, vprev, wrow, brow)


def kernel(x, edge_index, W1l, b1, W1r, W2l, b2, W2r, W3l, b3, W3r, Wlin, blin):
  f32 = jnp.float32

  z16 = jnp.zeros((_D, 16), f32)
  wa1 = jnp.concatenate([W1l.T, z16], axis=1)
  wa2 = jnp.concatenate([W2l.T, z16], axis=1)
  wa3 = jnp.concatenate([W3l.T, z16], axis=1)
  ones144 = jnp.zeros((1, 144), f32).at[0, _D].set(1.0)
  ones48 = jnp.zeros((1, 48), f32).at[0, 32].set(1.0)

  pad = _E_PAD - _E
  src_p = jnp.concatenate([edge_index[0], jnp.zeros((pad,), jnp.int32)])
  dst_p = jnp.concatenate(
      [edge_index[1], _N + (jnp.arange(pad, dtype=jnp.int32) % 16)])
  src3 = src_p.reshape(_NW * _CHUNKS, _SUBS, _CH_SUB)
  dst3 = dst_p.reshape(_NW * _CHUNKS, _SUBS, _CH_SUB)
  zeros144 = jnp.zeros((_N_PAD, 144), f32)
  zeros48 = jnp.zeros((_N_PAD, 48), f32)

  agg144 = _sc_agg(144)
  agg48 = _sc_agg(48)

  t1, v1 = _dense_first(x, wa1, ones144, W1r.T, b1[None, :])
  p1 = agg144(t1, src3, dst3, zeros144)
  t2, v2 = _dense_mid(p1, v1, wa2, ones144, W2r.T, b2[None, :])
  p2 = agg144(t2, src3, dst3, zeros144)
  t3, v3 = _dense_mid(p2, v2, wa3, ones48, W3r.T, b3[None, :])
  p3 = agg48(t3, src3, dst3, zeros48)
  return _dense_last(p3, v3, Wlin, blin[None, :])


# speedup vs baseline: 3.4297x; 3.4297x over previous
import functools

import jax
import jax.numpy as jnp
from jax import lax
from jax.experimental import pallas as pl
from jax.experimental.pallas import tpu as pltpu
from jax.experimental.pallas import tpu_sc as plsc

_N = 10000
_E = 320000
_D = 128

_NC = 2
_NS = 16
_NW = _NC * _NS
_CH_SUB = 128
_SUBS = 2
_CHUNK = _SUBS * _CH_SUB
_CHUNKS = 40
_EW = _CHUNK * _CHUNKS
_E_PAD = _EW * _NW
_N_PAD = 10112
_ZROWS = _N_PAD // _NS

_BLK = 1000


def _sc_agg(de):
  mesh = plsc.VectorSubcoreMesh(core_axis_name="c", subcore_axis_name="s")

  @functools.partial(
      pl.kernel,
      mesh=mesh,
      compiler_params=pltpu.CompilerParams(use_tc_tiling_on_sc=False),
      out_type=jax.ShapeDtypeStruct((_NC, _N_PAD, de), jnp.float32),
      scratch_types=[
          pltpu.VMEM((_CH_SUB, de), jnp.float32),
          pltpu.VMEM((_CH_SUB, de), jnp.float32),
          pltpu.VMEM((_SUBS, _CH_SUB), jnp.int32),
          pltpu.VMEM((_SUBS, _CH_SUB), jnp.int32),
          pltpu.VMEM_SHARED((_N_PAD, de), jnp.float32),
          pltpu.SemaphoreType.DMA,
      ],
  )
  def agg(table_h, src_h, dst_h, zeros_h, part_h,
          rows0, rows1, src_v, dst_v, acc, sem):
    rows = [rows0, rows1]
    c = lax.axis_index("c")
    s = lax.axis_index("s")
    wid = s * _NC + c

    pltpu.sync_copy(zeros_h.at[pl.ds(s * _ZROWS, _ZROWS)],
                    acc.at[pl.ds(s * _ZROWS, _ZROWS)])
    plsc.subcore_barrier()

    def body(i, carry):
      row = wid * _CHUNKS + i
      pltpu.sync_copy(src_h.at[row], src_v)
      pltpu.sync_copy(dst_h.at[row], dst_v)
      cps = [pltpu.async_copy(table_h.at[src_v.at[j]], rows[j], sem)
             for j in range(_SUBS)]
      for cp in cps:
        cp.wait()
      for j in range(_SUBS):
        pltpu.sync_copy(rows[j], acc.at[dst_v.at[j]], add=True)
      return carry

    lax.fori_loop(0, _CHUNKS, body, 0)
    plsc.subcore_barrier()

    pltpu.sync_copy(acc.at[pl.ds(s * _ZROWS, _ZROWS)],
                    part_h.at[c, pl.ds(s * _ZROWS, _ZROWS)])

  return agg


def _rep(shape):
  return pl.BlockSpec(shape, lambda i: tuple(0 for _ in shape))


def _dense_first(x, wa, ca, wb, cb):
  de, dv = wa.shape[1], wb.shape[1]

  def body(x_ref, wa_ref, ca_ref, wb_ref, cb_ref, t_ref, v_ref):
    xb = x_ref[...]
    t_ref[...] = jnp.dot(xb, wa_ref[...],
                         preferred_element_type=jnp.float32) + ca_ref[...]
    v_ref[...] = jnp.dot(xb, wb_ref[...],
                         preferred_element_type=jnp.float32) + cb_ref[...]

  return pl.pallas_call(
      body,
      grid=(_N // _BLK,),
      in_specs=[
          pl.BlockSpec((_BLK, _D), lambda i: (i, 0)),
          _rep(wa.shape), _rep(ca.shape), _rep(wb.shape), _rep(cb.shape),
      ],
      out_specs=[
          pl.BlockSpec((_BLK, de), lambda i: (i, 0)),
          pl.BlockSpec((_BLK, dv), lambda i: (i, 0)),
      ],
      out_shape=[
          jax.ShapeDtypeStruct((_N, de), jnp.float32),
          jax.ShapeDtypeStruct((_N, dv), jnp.float32),
      ],
  )(x, wa, ca, wb, cb)


def _dense_mid(p, vprev, wa, ca, wb, cb):
  din = vprev.shape[1]
  dpe = p.shape[2]
  de, dv = wa.shape[1], wb.shape[1]

  def body(p_ref, vp_ref, wa_ref, ca_ref, wb_ref, cb_ref, t_ref, v_ref):
    sall = p_ref[0] + p_ref[1]
    inv = 1.0 / jnp.maximum(sall[:, din:din + 1], 1.0)
    h = jnp.maximum(sall[:, :din] * inv + vp_ref[...], 0.0)
    t_ref[...] = jnp.dot(h, wa_ref[...],
                         preferred_element_type=jnp.float32) + ca_ref[...]
    v_ref[...] = jnp.dot(h, wb_ref[...],
                         preferred_element_type=jnp.float32) + cb_ref[...]

  return pl.pallas_call(
      body,
      grid=(_N // _BLK,),
      in_specs=[
          pl.BlockSpec((_NC, _BLK, dpe), lambda i: (0, i, 0)),
          pl.BlockSpec((_BLK, din), lambda i: (i, 0)),
          _rep(wa.shape), _rep(ca.shape), _rep(wb.shape), _rep(cb.shape),
      ],
      out_specs=[
          pl.BlockSpec((_BLK, de), lambda i: (i, 0)),
          pl.BlockSpec((_BLK, dv), lambda i: (i, 0)),
      ],
      out_shape=[
          jax.ShapeDtypeStruct((_N, de), jnp.float32),
          jax.ShapeDtypeStruct((_N, dv), jnp.float32),
      ],
  )(p, vprev, wa, ca, wb, cb)


def _dense_last(p, vprev, wrow, brow):
  din = vprev.shape[1]
  dpe = p.shape[2]

  def body(p_ref, vp_ref, w_ref, b_ref, o_ref):
    sall = p_ref[0] + p_ref[1]
    inv = 1.0 / jnp.maximum(sall[:, din:din + 1], 1.0)
    h = jnp.maximum(sall[:, :din] * inv + vp_ref[...], 0.0)
    o_ref[...] = jnp.sum(h * w_ref[...], axis=1, keepdims=True) + b_ref[...]

  return pl.pallas_call(
      body,
      grid=(_N // _BLK,),
      in_specs=[
          pl.BlockSpec((_NC, _BLK, dpe), lambda i: (0, i, 0)),
          pl.BlockSpec((_BLK, din), lambda i: (i, 0)),
          _rep(wrow.shape), _rep(brow.shape),
      ],
      out_specs=pl.BlockSpec((_BLK, 1), lambda i: (i, 0)),
      out_shape=jax.ShapeDtypeStruct((_N, 1), jnp.float32),
  )(p, vprev, wrow, brow)


def kernel(x, edge_index, W1l, b1, W1r, W2l, b2, W2r, W3l, b3, W3r, Wlin, blin):
  f32 = jnp.float32

  z16 = jnp.zeros((_D, 16), f32)
  wa1 = jnp.concatenate([W1l.T, z16], axis=1)
  wa2 = jnp.concatenate([W2l.T, z16], axis=1)
  wa3 = jnp.concatenate([W3l.T, z16], axis=1)
  ones144 = jnp.zeros((1, 144), f32).at[0, _D].set(1.0)
  ones48 = jnp.zeros((1, 48), f32).at[0, 32].set(1.0)

  pad = _E_PAD - _E
  src_p = jnp.concatenate([edge_index[0], jnp.zeros((pad,), jnp.int32)])
  dst_p = jnp.concatenate(
      [edge_index[1], _N + (jnp.arange(pad, dtype=jnp.int32) % 16)])
  src3 = src_p.reshape(_NW * _CHUNKS, _SUBS, _CH_SUB)
  dst3 = dst_p.reshape(_NW * _CHUNKS, _SUBS, _CH_SUB)
  zeros144 = jnp.zeros((_N_PAD, 144), f32)
  zeros48 = jnp.zeros((_N_PAD, 48), f32)

  agg144 = _sc_agg(144)
  agg48 = _sc_agg(48)

  t1, v1 = _dense_first(x, wa1, ones144, W1r.T, b1[None, :])
  p1 = agg144(t1, src3, dst3, zeros144)
  t2, v2 = _dense_mid(p1, v1, wa2, ones144, W2r.T, b2[None, :])
  p2 = agg144(t2, src3, dst3, zeros144)
  t3, v3 = _dense_mid(p2, v2, wa3, ones48, W3r.T, b3[None, :])
  p3 = agg48(t3, src3, dst3, zeros48)
  return _dense_last(p3, v3, Wlin, blin[None, :])


# speedup vs baseline: 3.8439x; 1.1208x over previous
import functools

import jax
import jax.numpy as jnp
from jax import lax
from jax.experimental import pallas as pl
from jax.experimental.pallas import tpu as pltpu
from jax.experimental.pallas import tpu_sc as plsc

_N = 10000
_E = 320000
_D = 128

_NC = 2
_NS = 16
_NW = _NC * _NS
_CH_SUB = 128
_PER_BLOCK = 10
_BLOCKS = 8
_EW = _CH_SUB * _PER_BLOCK * _BLOCKS
_E_PAD = _EW * _NW
_N_PAD = 10112
_ZROWS = _N_PAD // _NS

_BLK = 1000


def _sc_agg(de):
  mesh = plsc.VectorSubcoreMesh(core_axis_name="c", subcore_axis_name="s")

  @functools.partial(
      pl.kernel,
      mesh=mesh,
      compiler_params=pltpu.CompilerParams(use_tc_tiling_on_sc=False),
      out_type=jax.ShapeDtypeStruct((_NC, _N_PAD, de), jnp.float32),
      scratch_types=[
          pltpu.VMEM((_CH_SUB, de), jnp.float32),
          pltpu.VMEM((_CH_SUB, de), jnp.float32),
          pltpu.VMEM((_PER_BLOCK, _CH_SUB), jnp.int32),
          pltpu.VMEM((_PER_BLOCK, _CH_SUB), jnp.int32),
          pltpu.VMEM_SHARED((_N_PAD, de), jnp.float32),
          pltpu.SemaphoreType.DMA,
          pltpu.SemaphoreType.DMA,
      ],
  )
  def agg(table_h, src_h, dst_h, zeros_h, part_h,
          rows0, rows1, src_v, dst_v, acc, sem0, sem1):
    rows = [rows0, rows1]
    sems = [sem0, sem1]
    c = lax.axis_index("c")
    s = lax.axis_index("s")
    wid = s * _NC + c

    pltpu.sync_copy(zeros_h.at[pl.ds(s * _ZROWS, _ZROWS)],
                    acc.at[pl.ds(s * _ZROWS, _ZROWS)])
    plsc.subcore_barrier()

    def body(b, carry):
      row = wid * _BLOCKS + b
      pltpu.sync_copy(src_h.at[row], src_v)
      pltpu.sync_copy(dst_h.at[row], dst_v)
      cps = [None, None]
      cps[0] = pltpu.async_copy(table_h.at[src_v.at[0]], rows[0], sems[0])
      for d in range(_PER_BLOCK):
        p = d & 1
        if d + 1 < _PER_BLOCK:
          cps[1 - p] = pltpu.async_copy(
              table_h.at[src_v.at[d + 1]], rows[1 - p], sems[1 - p])
        cps[p].wait()
        pltpu.sync_copy(rows[p], acc.at[dst_v.at[d]], add=True)
      return carry

    lax.fori_loop(0, _BLOCKS, body, 0)
    plsc.subcore_barrier()

    pltpu.sync_copy(acc.at[pl.ds(s * _ZROWS, _ZROWS)],
                    part_h.at[c, pl.ds(s * _ZROWS, _ZROWS)])

  return agg


def _rep(shape):
  return pl.BlockSpec(shape, lambda i: tuple(0 for _ in shape))


def _dense_first(x, wa, ca, wb, cb):
  de, dv = wa.shape[1], wb.shape[1]

  def body(x_ref, wa_ref, ca_ref, wb_ref, cb_ref, t_ref, v_ref):
    xb = x_ref[...]
    t_ref[...] = jnp.dot(xb, wa_ref[...],
                         preferred_element_type=jnp.float32) + ca_ref[...]
    v_ref[...] = jnp.dot(xb, wb_ref[...],
                         preferred_element_type=jnp.float32) + cb_ref[...]

  return pl.pallas_call(
      body,
      grid=(_N // _BLK,),
      in_specs=[
          pl.BlockSpec((_BLK, _D), lambda i: (i, 0)),
          _rep(wa.shape), _rep(ca.shape), _rep(wb.shape), _rep(cb.shape),
      ],
      out_specs=[
          pl.BlockSpec((_BLK, de), lambda i: (i, 0)),
          pl.BlockSpec((_BLK, dv), lambda i: (i, 0)),
      ],
      out_shape=[
          jax.ShapeDtypeStruct((_N, de), jnp.float32),
          jax.ShapeDtypeStruct((_N, dv), jnp.float32),
      ],
  )(x, wa, ca, wb, cb)


def _dense_mid(p, vprev, wa, ca, wb, cb):
  din = vprev.shape[1]
  dpe = p.shape[2]
  de, dv = wa.shape[1], wb.shape[1]

  def body(p_ref, vp_ref, wa_ref, ca_ref, wb_ref, cb_ref, t_ref, v_ref):
    sall = p_ref[0] + p_ref[1]
    inv = 1.0 / jnp.maximum(sall[:, din:din + 1], 1.0)
    h = jnp.maximum(sall[:, :din] * inv + vp_ref[...], 0.0)
    t_ref[...] = jnp.dot(h, wa_ref[...],
                         preferred_element_type=jnp.float32) + ca_ref[...]
    v_ref[...] = jnp.dot(h, wb_ref[...],
                         preferred_element_type=jnp.float32) + cb_ref[...]

  return pl.pallas_call(
      body,
      grid=(_N // _BLK,),
      in_specs=[
          pl.BlockSpec((_NC, _BLK, dpe), lambda i: (0, i, 0)),
          pl.BlockSpec((_BLK, din), lambda i: (i, 0)),
          _rep(wa.shape), _rep(ca.shape), _rep(wb.shape), _rep(cb.shape),
      ],
      out_specs=[
          pl.BlockSpec((_BLK, de), lambda i: (i, 0)),
          pl.BlockSpec((_BLK, dv), lambda i: (i, 0)),
      ],
      out_shape=[
          jax.ShapeDtypeStruct((_N, de), jnp.float32),
          jax.ShapeDtypeStruct((_N, dv), jnp.float32),
      ],
  )(p, vprev, wa, ca, wb, cb)


def _dense_last(p, vprev, wrow, brow):
  din = vprev.shape[1]
  dpe = p.shape[2]

  def body(p_ref, vp_ref, w_ref, b_ref, o_ref):
    sall = p_ref[0] + p_ref[1]
    inv = 1.0 / jnp.maximum(sall[:, din:din + 1], 1.0)
    h = jnp.maximum(sall[:, :din] * inv + vp_ref[...], 0.0)
    o_ref[...] = jnp.sum(h * w_ref[...], axis=1, keepdims=True) + b_ref[...]

  return pl.pallas_call(
      body,
      grid=(_N // _BLK,),
      in_specs=[
          pl.BlockSpec((_NC, _BLK, dpe), lambda i: (0, i, 0)),
          pl.BlockSpec((_BLK, din), lambda i: (i, 0)),
          _rep(wrow.shape), _rep(brow.shape),
      ],
      out_specs=pl.BlockSpec((_BLK, 1), lambda i: (i, 0)),
      out_shape=jax.ShapeDtypeStruct((_N, 1), jnp.float32),
  )(p, vprev, wrow, brow)


def kernel(x, edge_index, W1l, b1, W1r, W2l, b2, W2r, W3l, b3, W3r, Wlin, blin):
  f32 = jnp.float32

  z16 = jnp.zeros((_D, 16), f32)
  wa1 = jnp.concatenate([W1l.T, z16], axis=1)
  wa2 = jnp.concatenate([W2l.T, z16], axis=1)
  wa3 = jnp.concatenate([W3l.T, z16], axis=1)
  ones144 = jnp.zeros((1, 144), f32).at[0, _D].set(1.0)
  ones48 = jnp.zeros((1, 48), f32).at[0, 32].set(1.0)

  pad = _E_PAD - _E
  src_p = jnp.concatenate([edge_index[0], jnp.zeros((pad,), jnp.int32)])
  dst_p = jnp.concatenate(
      [edge_index[1], _N + (jnp.arange(pad, dtype=jnp.int32) % 16)])
  src3 = src_p.reshape(_NW * _BLOCKS, _PER_BLOCK, _CH_SUB)
  dst3 = dst_p.reshape(_NW * _BLOCKS, _PER_BLOCK, _CH_SUB)
  zeros144 = jnp.zeros((_N_PAD, 144), f32)
  zeros48 = jnp.zeros((_N_PAD, 48), f32)

  agg144 = _sc_agg(144)
  agg48 = _sc_agg(48)

  t1, v1 = _dense_first(x, wa1, ones144, W1r.T, b1[None, :])
  p1 = agg144(t1, src3, dst3, zeros144)
  t2, v2 = _dense_mid(p1, v1, wa2, ones144, W2r.T, b2[None, :])
  p2 = agg144(t2, src3, dst3, zeros144)
  t3, v3 = _dense_mid(p2, v2, wa3, ones48, W3r.T, b3[None, :])
  p3 = agg48(t3, src3, dst3, zeros48)
  return _dense_last(p3, v3, Wlin, blin[None, :])


# speedup vs baseline: 10.1906x; 2.6511x over previous
import functools

import jax
import jax.numpy as jnp
from jax import lax
from jax.experimental import pallas as pl
from jax.experimental.pallas import tpu as pltpu
from jax.experimental.pallas import tpu_sc as plsc

_N = 10000
_E = 320000
_D = 128

_NC = 2
_NS = 16
_NW = _NC * _NS
_CH_SUB = 128
_PER_BLOCK = 10
_BLOCKS = 8
_EW = _CH_SUB * _PER_BLOCK * _BLOCKS
_REAL_LAST_BLOCKS = 2
_E_PAD = _EW * _NW
_N_PAD = 10112
_ZROWS = _N_PAD // _NS

_BLK = 1000


def _sc_agg(de):
  mesh = plsc.VectorSubcoreMesh(core_axis_name="c", subcore_axis_name="s")

  @functools.partial(
      pl.kernel,
      mesh=mesh,
      compiler_params=pltpu.CompilerParams(use_tc_tiling_on_sc=False),
      out_type=jax.ShapeDtypeStruct((_NC, _N_PAD, de), jnp.float32),
      scratch_types=[
          pltpu.VMEM((_CH_SUB, de), jnp.float32),
          pltpu.VMEM((_CH_SUB, de), jnp.float32),
          pltpu.VMEM((_PER_BLOCK, _CH_SUB), jnp.int32),
          pltpu.VMEM((_PER_BLOCK, _CH_SUB), jnp.int32),
          pltpu.VMEM_SHARED((_N_PAD, de), jnp.float32),
          pltpu.SemaphoreType.DMA,
          pltpu.SemaphoreType.DMA,
      ],
  )
  def agg(table_h, src_h, dst_h, zeros_h, part_h,
          rows0, rows1, src_v, dst_v, acc, sem0, sem1):
    rows = [rows0, rows1]
    sems = [sem0, sem1]
    c = lax.axis_index("c")
    s = lax.axis_index("s")
    wid = s * _NC + c

    pltpu.sync_copy(zeros_h.at[pl.ds(s * _ZROWS, _ZROWS)],
                    acc.at[pl.ds(s * _ZROWS, _ZROWS)])
    plsc.subcore_barrier()

    nb = jnp.where(wid == _NW - 1, _REAL_LAST_BLOCKS, _BLOCKS)

    def body(b, carry):
      row = wid * _BLOCKS + b
      pltpu.sync_copy(src_h.at[row], src_v)
      pltpu.sync_copy(dst_h.at[row], dst_v)
      cps = [None, None]
      cps[0] = pltpu.async_copy(table_h.at[src_v.at[0]], rows[0], sems[0])
      for d in range(_PER_BLOCK):
        p = d & 1
        if d + 1 < _PER_BLOCK:
          cps[1 - p] = pltpu.async_copy(
              table_h.at[src_v.at[d + 1]], rows[1 - p], sems[1 - p])
        cps[p].wait()
        pltpu.sync_copy(rows[p], acc.at[dst_v.at[d]], add=True)
      return carry

    lax.fori_loop(0, nb, body, 0)
    plsc.subcore_barrier()

    pltpu.sync_copy(acc.at[pl.ds(s * _ZROWS, _ZROWS)],
                    part_h.at[c, pl.ds(s * _ZROWS, _ZROWS)])

  return agg


def _rep(shape):
  return pl.BlockSpec(shape, lambda i: tuple(0 for _ in shape))


def _dense_first(x, wa, ca, wb, cb):
  de, dv = wa.shape[1], wb.shape[1]

  def body(x_ref, wa_ref, ca_ref, wb_ref, cb_ref, t_ref, v_ref):
    xb = x_ref[...]
    t_ref[...] = jnp.dot(xb, wa_ref[...],
                         preferred_element_type=jnp.float32) + ca_ref[...]
    v_ref[...] = jnp.dot(xb, wb_ref[...],
                         preferred_element_type=jnp.float32) + cb_ref[...]

  return pl.pallas_call(
      body,
      grid=(_N // _BLK,),
      in_specs=[
          pl.BlockSpec((_BLK, _D), lambda i: (i, 0)),
          _rep(wa.shape), _rep(ca.shape), _rep(wb.shape), _rep(cb.shape),
      ],
      out_specs=[
          pl.BlockSpec((_BLK, de), lambda i: (i, 0)),
          pl.BlockSpec((_BLK, dv), lambda i: (i, 0)),
      ],
      out_shape=[
          jax.ShapeDtypeStruct((_N, de), jnp.float32),
          jax.ShapeDtypeStruct((_N, dv), jnp.float32),
      ],
  )(x, wa, ca, wb, cb)


def _dense_mid(p, vprev, wa, ca, wb, cb):
  din = vprev.shape[1]
  dpe = p.shape[2]
  de, dv = wa.shape[1], wb.shape[1]

  def body(p_ref, vp_ref, wa_ref, ca_ref, wb_ref, cb_ref, t_ref, v_ref):
    sall = p_ref[0] + p_ref[1]
    inv = 1.0 / jnp.maximum(sall[:, din:din + 1], 1.0)
    h = jnp.maximum(sall[:, :din] * inv + vp_ref[...], 0.0)
    t_ref[...] = jnp.dot(h, wa_ref[...],
                         preferred_element_type=jnp.float32) + ca_ref[...]
    v_ref[...] = jnp.dot(h, wb_ref[...],
                         preferred_element_type=jnp.float32) + cb_ref[...]

  return pl.pallas_call(
      body,
      grid=(_N // _BLK,),
      in_specs=[
          pl.BlockSpec((_NC, _BLK, dpe), lambda i: (0, i, 0)),
          pl.BlockSpec((_BLK, din), lambda i: (i, 0)),
          _rep(wa.shape), _rep(ca.shape), _rep(wb.shape), _rep(cb.shape),
      ],
      out_specs=[
          pl.BlockSpec((_BLK, de), lambda i: (i, 0)),
          pl.BlockSpec((_BLK, dv), lambda i: (i, 0)),
      ],
      out_shape=[
          jax.ShapeDtypeStruct((_N, de), jnp.float32),
          jax.ShapeDtypeStruct((_N, dv), jnp.float32),
      ],
  )(p, vprev, wa, ca, wb, cb)


def _dense_last(p, vprev, wrow, brow):
  din = vprev.shape[1]
  dpe = p.shape[2]

  def body(p_ref, vp_ref, w_ref, b_ref, o_ref):
    sall = p_ref[0] + p_ref[1]
    inv = 1.0 / jnp.maximum(sall[:, din:din + 1], 1.0)
    h = jnp.maximum(sall[:, :din] * inv + vp_ref[...], 0.0)
    o_ref[...] = jnp.sum(h * w_ref[...], axis=1, keepdims=True) + b_ref[...]

  return pl.pallas_call(
      body,
      grid=(_N // _BLK,),
      in_specs=[
          pl.BlockSpec((_NC, _BLK, dpe), lambda i: (0, i, 0)),
          pl.BlockSpec((_BLK, din), lambda i: (i, 0)),
          _rep(wrow.shape), _rep(brow.shape),
      ],
      out_specs=pl.BlockSpec((_BLK, 1), lambda i: (i, 0)),
      out_shape=jax.ShapeDtypeStruct((_N, 1), jnp.float32),
  )(p, vprev, wrow, brow)


def kernel(x, edge_index, W1l, b1, W1r, W2l, b2, W2r, W3l, b3, W3r, Wlin, blin):
  f32 = jnp.float32

  z16 = jnp.zeros((_D, 16), f32)
  wa1 = jnp.concatenate([W1l.T, z16], axis=1)
  wa2 = jnp.concatenate([W2l.T, z16], axis=1)
  wa3 = jnp.concatenate([W3l.T, z16], axis=1)
  ones144 = jnp.zeros((1, 144), f32).at[0, _D].set(1.0)
  ones48 = jnp.zeros((1, 48), f32).at[0, 32].set(1.0)

  pad = _E_PAD - _E
  src_p = jnp.concatenate([edge_index[0], jnp.zeros((pad,), jnp.int32)])
  dst_p = jnp.concatenate(
      [edge_index[1], _N + (jnp.arange(pad, dtype=jnp.int32) % 16)])
  src3 = src_p.reshape(_NW * _BLOCKS, _PER_BLOCK, _CH_SUB)
  dst3 = dst_p.reshape(_NW * _BLOCKS, _PER_BLOCK, _CH_SUB)
  zeros144 = jnp.zeros((_N_PAD, 144), f32)
  zeros48 = jnp.zeros((_N_PAD, 48), f32)

  agg144 = _sc_agg(144)
  agg48 = _sc_agg(48)

  t1, v1 = _dense_first(x, wa1, ones144, W1r.T, b1[None, :])
  p1 = agg144(t1, src3, dst3, zeros144)
  t2, v2 = _dense_mid(p1, v1, wa2, ones144, W2r.T, b2[None, :])
  p2 = agg144(t2, src3, dst3, zeros144)
  t3, v3 = _dense_mid(p2, v2, wa3, ones48, W3r.T, b3[None, :])
  p3 = agg48(t3, src3, dst3, zeros48)
  return _dense_last(p3, v3, Wlin, blin[None, :])


# speedup vs baseline: 11.3933x; 1.1180x over previous
import functools

import jax
import jax.numpy as jnp
from jax import lax
from jax.experimental import pallas as pl
from jax.experimental.pallas import tpu as pltpu
from jax.experimental.pallas import tpu_sc as plsc

_N = 10000
_E = 320000
_D = 128

_NC = 2
_NS = 16
_NW = _NC * _NS
_CH_SUB = 128
_PER_BLOCK = 10
_BLOCKS = 8
_EW = _CH_SUB * _PER_BLOCK * _BLOCKS
_REAL_LAST_BLOCKS = 2
_E_PAD = _EW * _NW
_N_PAD = 10112
_ZROWS = _N_PAD // _NS

_BLK = 1000


def _sc_agg(de):
  mesh = plsc.VectorSubcoreMesh(core_axis_name="c", subcore_axis_name="s")

  @functools.partial(
      pl.kernel,
      mesh=mesh,
      compiler_params=pltpu.CompilerParams(use_tc_tiling_on_sc=False),
      out_type=jax.ShapeDtypeStruct((_NC, _N_PAD, de), jnp.float32),
      scratch_types=[
          pltpu.VMEM((_CH_SUB, de), jnp.float32),
          pltpu.VMEM((_CH_SUB, de), jnp.float32),
          pltpu.VMEM((_PER_BLOCK, _CH_SUB), jnp.int32),
          pltpu.VMEM((_PER_BLOCK, _CH_SUB), jnp.int32),
          pltpu.VMEM_SHARED((_N_PAD, de), jnp.float32),
          pltpu.SemaphoreType.DMA,
          pltpu.SemaphoreType.DMA,
      ],
  )
  def agg(table_h, src_h, dst_h, zeros_h, part_h,
          rows0, rows1, src_v, dst_v, acc, sem0, sem1):
    rows = [rows0, rows1]
    sems = [sem0, sem1]
    c = lax.axis_index("c")
    s = lax.axis_index("s")
    wid = s * _NC + c

    pltpu.sync_copy(zeros_h.at[pl.ds(s * _ZROWS, _ZROWS)],
                    acc.at[pl.ds(s * _ZROWS, _ZROWS)])
    plsc.subcore_barrier()

    nb = jnp.where(wid == _NW - 1, _REAL_LAST_BLOCKS, _BLOCKS)

    def body(b, carry):
      row = wid * _BLOCKS + b
      pltpu.sync_copy(src_h.at[row], src_v)
      pltpu.sync_copy(dst_h.at[row], dst_v)
      cps = [None, None]
      cps[0] = pltpu.async_copy(table_h.at[src_v.at[0]], rows[0], sems[0])
      for d in range(_PER_BLOCK):
        p = d & 1
        if d + 1 < _PER_BLOCK:
          cps[1 - p] = pltpu.async_copy(
              table_h.at[src_v.at[d + 1]], rows[1 - p], sems[1 - p])
        cps[p].wait()
        pltpu.sync_copy(rows[p], acc.at[dst_v.at[d]], add=True)
      return carry

    lax.fori_loop(0, nb, body, 0)
    plsc.subcore_barrier()

    pltpu.sync_copy(acc.at[pl.ds(s * _ZROWS, _ZROWS)],
                    part_h.at[c, pl.ds(s * _ZROWS, _ZROWS)])

  return agg


def _rep(shape):
  return pl.BlockSpec(shape, lambda i: tuple(0 for _ in shape))


def _dense_first(x, wa, ca, wb, cb):
  de, dv = wa.shape[1], wb.shape[1]

  def body(x_ref, wa_ref, ca_ref, wb_ref, cb_ref, t_ref, v_ref):
    xb = x_ref[...]
    t_ref[...] = jnp.dot(xb, wa_ref[...],
                         preferred_element_type=jnp.float32) + ca_ref[...]
    v_ref[...] = jnp.dot(xb, wb_ref[...],
                         preferred_element_type=jnp.float32) + cb_ref[...]

  return pl.pallas_call(
      body,
      grid=(_N // _BLK,),
      in_specs=[
          pl.BlockSpec((_BLK, _D), lambda i: (i, 0)),
          _rep(wa.shape), _rep(ca.shape), _rep(wb.shape), _rep(cb.shape),
      ],
      out_specs=[
          pl.BlockSpec((_BLK, de), lambda i: (i, 0)),
          pl.BlockSpec((_BLK, dv), lambda i: (i, 0)),
      ],
      out_shape=[
          jax.ShapeDtypeStruct((_N, de), jnp.float32),
          jax.ShapeDtypeStruct((_N, dv), jnp.float32),
      ],
  )(x, wa, ca, wb, cb)


def _dense_mid_cnt(p, vprev, wa, wb, cb):
  din = vprev.shape[1]
  dpe = p.shape[2]
  de, dv = wa.shape[1], wb.shape[1]

  def body(p_ref, vp_ref, wa_ref, wb_ref, cb_ref, t_ref, v_ref, iv_ref):
    sall = p_ref[0] + p_ref[1]
    inv = 1.0 / jnp.maximum(sall[:, din:din + 1], 1.0)
    h = jnp.maximum(sall[:, :din] * inv + vp_ref[...], 0.0)
    t_ref[...] = jnp.dot(h, wa_ref[...], preferred_element_type=jnp.float32)
    v_ref[...] = jnp.dot(h, wb_ref[...],
                         preferred_element_type=jnp.float32) + cb_ref[...]
    iv_ref[...] = jnp.broadcast_to(inv, (inv.shape[0], 16))

  return pl.pallas_call(
      body,
      grid=(_N // _BLK,),
      in_specs=[
          pl.BlockSpec((_NC, _BLK, dpe), lambda i: (0, i, 0)),
          pl.BlockSpec((_BLK, din), lambda i: (i, 0)),
          _rep(wa.shape), _rep(wb.shape), _rep(cb.shape),
      ],
      out_specs=[
          pl.BlockSpec((_BLK, de), lambda i: (i, 0)),
          pl.BlockSpec((_BLK, dv), lambda i: (i, 0)),
          pl.BlockSpec((_BLK, 16), lambda i: (i, 0)),
      ],
      out_shape=[
          jax.ShapeDtypeStruct((_N, de), jnp.float32),
          jax.ShapeDtypeStruct((_N, dv), jnp.float32),
          jax.ShapeDtypeStruct((_N, 16), jnp.float32),
      ],
  )(p, vprev, wa, wb, cb)


def _dense_mid_inv(p, inv16, vprev, wa, wb, cb):
  din = vprev.shape[1]
  dpe = p.shape[2]
  de, dv = wa.shape[1], wb.shape[1]

  def body(p_ref, iv_ref, vp_ref, wa_ref, wb_ref, cb_ref, t_ref, v_ref):
    sall = p_ref[0] + p_ref[1]
    inv = iv_ref[...][:, :1]
    h = jnp.maximum(sall[:, :din] * inv + vp_ref[...], 0.0)
    t_ref[...] = jnp.dot(h, wa_ref[...], preferred_element_type=jnp.float32)
    v_ref[...] = jnp.dot(h, wb_ref[...],
                         preferred_element_type=jnp.float32) + cb_ref[...]

  return pl.pallas_call(
      body,
      grid=(_N // _BLK,),
      in_specs=[
          pl.BlockSpec((_NC, _BLK, dpe), lambda i: (0, i, 0)),
          pl.BlockSpec((_BLK, 16), lambda i: (i, 0)),
          pl.BlockSpec((_BLK, din), lambda i: (i, 0)),
          _rep(wa.shape), _rep(wb.shape), _rep(cb.shape),
      ],
      out_specs=[
          pl.BlockSpec((_BLK, de), lambda i: (i, 0)),
          pl.BlockSpec((_BLK, dv), lambda i: (i, 0)),
      ],
      out_shape=[
          jax.ShapeDtypeStruct((_N, de), jnp.float32),
          jax.ShapeDtypeStruct((_N, dv), jnp.float32),
      ],
  )(p, inv16, vprev, wa, wb, cb)


def _dense_last(p, inv16, vprev, wrow, brow):
  din = vprev.shape[1]
  dpe = p.shape[2]

  def body(p_ref, iv_ref, vp_ref, w_ref, b_ref, o_ref):
    sall = p_ref[0] + p_ref[1]
    inv = iv_ref[...][:, :1]
    h = jnp.maximum(sall[:, :din] * inv + vp_ref[...], 0.0)
    o_ref[...] = jnp.sum(h * w_ref[...], axis=1, keepdims=True) + b_ref[...]

  return pl.pallas_call(
      body,
      grid=(_N // _BLK,),
      in_specs=[
          pl.BlockSpec((_NC, _BLK, dpe), lambda i: (0, i, 0)),
          pl.BlockSpec((_BLK, 16), lambda i: (i, 0)),
          pl.BlockSpec((_BLK, din), lambda i: (i, 0)),
          _rep(wrow.shape), _rep(brow.shape),
      ],
      out_specs=pl.BlockSpec((_BLK, 1), lambda i: (i, 0)),
      out_shape=jax.ShapeDtypeStruct((_N, 1), jnp.float32),
  )(p, inv16, vprev, wrow, brow)


def kernel(x, edge_index, W1l, b1, W1r, W2l, b2, W2r, W3l, b3, W3r, Wlin, blin):
  f32 = jnp.float32

  z16 = jnp.zeros((_D, 16), f32)
  wa1 = jnp.concatenate([W1l.T, z16], axis=1)
  ones144 = jnp.zeros((1, 144), f32).at[0, _D].set(1.0)

  pad = _E_PAD - _E
  src_p = jnp.concatenate([edge_index[0], jnp.zeros((pad,), jnp.int32)])
  dst_p = jnp.concatenate(
      [edge_index[1], _N + (jnp.arange(pad, dtype=jnp.int32) % 16)])
  src3 = src_p.reshape(_NW * _BLOCKS, _PER_BLOCK, _CH_SUB)
  dst3 = dst_p.reshape(_NW * _BLOCKS, _PER_BLOCK, _CH_SUB)
  zeros144 = jnp.zeros((_N_PAD, 144), f32)
  zeros128 = jnp.zeros((_N_PAD, 128), f32)
  zeros32 = jnp.zeros((_N_PAD, 32), f32)

  t1, v1 = _dense_first(x, wa1, ones144, W1r.T, b1[None, :])
  p1 = _sc_agg(144)(t1, src3, dst3, zeros144)
  t2, v2, inv16 = _dense_mid_cnt(p1, v1, W2l.T, W2r.T, b2[None, :])
  p2 = _sc_agg(128)(t2, src3, dst3, zeros128)
  t3, v3 = _dense_mid_inv(p2, inv16, v2, W3l.T, W3r.T, b3[None, :])
  p3 = _sc_agg(32)(t3, src3, dst3, zeros32)
  return _dense_last(p3, inv16, v3, Wlin, blin[None, :])


# speedup vs baseline: 12.2372x; 1.0741x over previous
import functools

import jax
import jax.numpy as jnp
from jax import lax
from jax.experimental import pallas as pl
from jax.experimental.pallas import tpu as pltpu
from jax.experimental.pallas import tpu_sc as plsc

_N = 10000
_E = 320000
_D = 128

_NC = 2
_NS = 16
_NW = _NC * _NS
_CH_SUB = 128
_PER_BLOCK = 10
_BLOCKS = 8
_EW = _CH_SUB * _PER_BLOCK * _BLOCKS
_REAL_LAST_BLOCKS = 2
_E_PAD = _EW * _NW
_N_PAD = 10112
_ZROWS = _N_PAD // _NS

_BLK = 1000


def _sc_agg(de, with_cnt=False):
  mesh = plsc.VectorSubcoreMesh(core_axis_name="c", subcore_axis_name="s")

  out_type = jax.ShapeDtypeStruct((_NC, _N_PAD, de), jnp.float32)
  if with_cnt:
    out_type = [out_type,
                jax.ShapeDtypeStruct((_NC, _N_PAD, 16), jnp.float32)]
  scratch = [
      pltpu.VMEM((_CH_SUB, de), jnp.float32),
      pltpu.VMEM((_CH_SUB, de), jnp.float32),
      pltpu.VMEM((_PER_BLOCK, _CH_SUB), jnp.int32),
      pltpu.VMEM((_PER_BLOCK, _CH_SUB), jnp.int32),
      pltpu.VMEM_SHARED((_N_PAD, de), jnp.float32),
      pltpu.SemaphoreType.DMA,
      pltpu.SemaphoreType.DMA,
  ]
  if with_cnt:
    scratch += [pltpu.VMEM((_CH_SUB, 16), jnp.float32),
                pltpu.VMEM_SHARED((_N_PAD, 16), jnp.float32)]

  @functools.partial(
      pl.kernel,
      mesh=mesh,
      compiler_params=pltpu.CompilerParams(use_tc_tiling_on_sc=False),
      out_type=out_type,
      scratch_types=scratch,
  )
  def agg(table_h, src_h, dst_h, zeros_h, *rest):
    if with_cnt:
      (ones_h, zeros16_h, part_h, cnt_h,
       rows0, rows1, src_v, dst_v, acc, sem0, sem1, ones_v, cacc) = rest
    else:
      (part_h, rows0, rows1, src_v, dst_v, acc, sem0, sem1) = rest
    rows = [rows0, rows1]
    sems = [sem0, sem1]
    c = lax.axis_index("c")
    s = lax.axis_index("s")
    wid = s * _NC + c

    pltpu.sync_copy(zeros_h.at[pl.ds(s * _ZROWS, _ZROWS)],
                    acc.at[pl.ds(s * _ZROWS, _ZROWS)])
    if with_cnt:
      pltpu.sync_copy(zeros16_h.at[pl.ds(s * _ZROWS, _ZROWS)],
                      cacc.at[pl.ds(s * _ZROWS, _ZROWS)])
      pltpu.sync_copy(ones_h, ones_v)
    plsc.subcore_barrier()

    nb = jnp.where(wid == _NW - 1, _REAL_LAST_BLOCKS, _BLOCKS)

    def body(b, carry):
      row = wid * _BLOCKS + b
      pltpu.sync_copy(src_h.at[row], src_v)
      pltpu.sync_copy(dst_h.at[row], dst_v)
      cps = [None, None]
      cps[0] = pltpu.async_copy(table_h.at[src_v.at[0]], rows[0], sems[0])
      for d in range(_PER_BLOCK):
        p = d & 1
        if d + 1 < _PER_BLOCK:
          cps[1 - p] = pltpu.async_copy(
              table_h.at[src_v.at[d + 1]], rows[1 - p], sems[1 - p])
        cps[p].wait()
        pltpu.sync_copy(rows[p], acc.at[dst_v.at[d]], add=True)
        if with_cnt:
          pltpu.sync_copy(ones_v, cacc.at[dst_v.at[d]], add=True)
      return carry

    lax.fori_loop(0, nb, body, 0)
    plsc.subcore_barrier()

    pltpu.sync_copy(acc.at[pl.ds(s * _ZROWS, _ZROWS)],
                    part_h.at[c, pl.ds(s * _ZROWS, _ZROWS)])
    if with_cnt:
      pltpu.sync_copy(cacc.at[pl.ds(s * _ZROWS, _ZROWS)],
                      cnt_h.at[c, pl.ds(s * _ZROWS, _ZROWS)])

  return agg


def _rep(shape):
  return pl.BlockSpec(shape, lambda i: tuple(0 for _ in shape))


def _dense_first(x, wa, wb, cb):
  de, dv = wa.shape[1], wb.shape[1]

  def body(x_ref, wa_ref, wb_ref, cb_ref, t_ref, v_ref):
    xb = x_ref[...]
    t_ref[...] = jnp.dot(xb, wa_ref[...], preferred_element_type=jnp.float32)
    v_ref[...] = jnp.dot(xb, wb_ref[...],
                         preferred_element_type=jnp.float32) + cb_ref[...]

  return pl.pallas_call(
      body,
      grid=(_N // _BLK,),
      in_specs=[
          pl.BlockSpec((_BLK, _D), lambda i: (i, 0)),
          _rep(wa.shape), _rep(wb.shape), _rep(cb.shape),
      ],
      out_specs=[
          pl.BlockSpec((_BLK, de), lambda i: (i, 0)),
          pl.BlockSpec((_BLK, dv), lambda i: (i, 0)),
      ],
      out_shape=[
          jax.ShapeDtypeStruct((_N, de), jnp.float32),
          jax.ShapeDtypeStruct((_N, dv), jnp.float32),
      ],
  )(x, wa, wb, cb)


def _dense_mid_cnt(p, cnt, vprev, wa, wb, cb):
  din = vprev.shape[1]
  dpe = p.shape[2]
  de, dv = wa.shape[1], wb.shape[1]

  def body(p_ref, c_ref, vp_ref, wa_ref, wb_ref, cb_ref,
           t_ref, v_ref, iv_ref):
    sall = p_ref[0] + p_ref[1]
    inv = 1.0 / jnp.maximum(c_ref[0][:, :1] + c_ref[1][:, :1], 1.0)
    h = jnp.maximum(sall * inv + vp_ref[...], 0.0)
    t_ref[...] = jnp.dot(h, wa_ref[...], preferred_element_type=jnp.float32)
    v_ref[...] = jnp.dot(h, wb_ref[...],
                         preferred_element_type=jnp.float32) + cb_ref[...]
    iv_ref[...] = jnp.broadcast_to(inv, (inv.shape[0], 16))

  return pl.pallas_call(
      body,
      grid=(_N // _BLK,),
      in_specs=[
          pl.BlockSpec((_NC, _BLK, dpe), lambda i: (0, i, 0)),
          pl.BlockSpec((_NC, _BLK, 16), lambda i: (0, i, 0)),
          pl.BlockSpec((_BLK, din), lambda i: (i, 0)),
          _rep(wa.shape), _rep(wb.shape), _rep(cb.shape),
      ],
      out_specs=[
          pl.BlockSpec((_BLK, de), lambda i: (i, 0)),
          pl.BlockSpec((_BLK, dv), lambda i: (i, 0)),
          pl.BlockSpec((_BLK, 16), lambda i: (i, 0)),
      ],
      out_shape=[
          jax.ShapeDtypeStruct((_N, de), jnp.float32),
          jax.ShapeDtypeStruct((_N, dv), jnp.float32),
          jax.ShapeDtypeStruct((_N, 16), jnp.float32),
      ],
  )(p, cnt, vprev, wa, wb, cb)


def _dense_mid_inv(p, inv16, vprev, wa, wb, cb):
  din = vprev.shape[1]
  dpe = p.shape[2]
  de, dv = wa.shape[1], wb.shape[1]

  def body(p_ref, iv_ref, vp_ref, wa_ref, wb_ref, cb_ref, t_ref, v_ref):
    sall = p_ref[0] + p_ref[1]
    inv = iv_ref[...][:, :1]
    h = jnp.maximum(sall[:, :din] * inv + vp_ref[...], 0.0)
    t_ref[...] = jnp.dot(h, wa_ref[...], preferred_element_type=jnp.float32)
    v_ref[...] = jnp.dot(h, wb_ref[...],
                         preferred_element_type=jnp.float32) + cb_ref[...]

  return pl.pallas_call(
      body,
      grid=(_N // _BLK,),
      in_specs=[
          pl.BlockSpec((_NC, _BLK, dpe), lambda i: (0, i, 0)),
          pl.BlockSpec((_BLK, 16), lambda i: (i, 0)),
          pl.BlockSpec((_BLK, din), lambda i: (i, 0)),
          _rep(wa.shape), _rep(wb.shape), _rep(cb.shape),
      ],
      out_specs=[
          pl.BlockSpec((_BLK, de), lambda i: (i, 0)),
          pl.BlockSpec((_BLK, dv), lambda i: (i, 0)),
      ],
      out_shape=[
          jax.ShapeDtypeStruct((_N, de), jnp.float32),
          jax.ShapeDtypeStruct((_N, dv), jnp.float32),
      ],
  )(p, inv16, vprev, wa, wb, cb)


def _dense_last(p, inv16, vprev, wrow, brow):
  din = vprev.shape[1]
  dpe = p.shape[2]

  def body(p_ref, iv_ref, vp_ref, w_ref, b_ref, o_ref):
    sall = p_ref[0] + p_ref[1]
    inv = iv_ref[...][:, :1]
    h = jnp.maximum(sall[:, :din] * inv + vp_ref[...], 0.0)
    o_ref[...] = jnp.sum(h * w_ref[...], axis=1, keepdims=True) + b_ref[...]

  return pl.pallas_call(
      body,
      grid=(_N // _BLK,),
      in_specs=[
          pl.BlockSpec((_NC, _BLK, dpe), lambda i: (0, i, 0)),
          pl.BlockSpec((_BLK, 16), lambda i: (i, 0)),
          pl.BlockSpec((_BLK, din), lambda i: (i, 0)),
          _rep(wrow.shape), _rep(brow.shape),
      ],
      out_specs=pl.BlockSpec((_BLK, 1), lambda i: (i, 0)),
      out_shape=jax.ShapeDtypeStruct((_N, 1), jnp.float32),
  )(p, inv16, vprev, wrow, brow)


def kernel(x, edge_index, W1l, b1, W1r, W2l, b2, W2r, W3l, b3, W3r, Wlin, blin):
  f32 = jnp.float32

  pad = _E_PAD - _E
  src_p = jnp.concatenate([edge_index[0], jnp.zeros((pad,), jnp.int32)])
  dst_p = jnp.concatenate(
      [edge_index[1], _N + (jnp.arange(pad, dtype=jnp.int32) % 16)])
  src3 = src_p.reshape(_NW * _BLOCKS, _PER_BLOCK, _CH_SUB)
  dst3 = dst_p.reshape(_NW * _BLOCKS, _PER_BLOCK, _CH_SUB)
  zeros128 = jnp.zeros((_N_PAD, 128), f32)
  zeros16 = jnp.zeros((_N_PAD, 16), f32)
  zeros32 = jnp.zeros((_N_PAD, 32), f32)
  ones16 = jnp.ones((_CH_SUB, 16), f32)

  t1, v1 = _dense_first(x, W1l.T, W1r.T, b1[None, :])
  p1, cnt = _sc_agg(128, with_cnt=True)(
      t1, src3, dst3, zeros128, ones16, zeros16)
  t2, v2, inv16 = _dense_mid_cnt(p1, cnt, v1, W2l.T, W2r.T, b2[None, :])
  p2 = _sc_agg(128)(t2, src3, dst3, zeros128)
  t3, v3 = _dense_mid_inv(p2, inv16, v2, W3l.T, W3r.T, b3[None, :])
  p3 = _sc_agg(32)(t3, src3, dst3, zeros32)
  return _dense_last(p3, inv16, v3, Wlin, blin[None, :])
